# baseline (speedup 1.0000x reference)
import jax
import jax.numpy as jnp
from jax.experimental import pallas as pl
from jax.experimental.pallas import tpu as pltpu

_O = 10


_NS = 4


def _copies(u_hbm, scrA, scrB, sems, a, i, slot, bB):
    cs = []
    p = bB // _NS
    for k in range(_NS):
        r0 = i * bB + k * p
        cs.append(pltpu.make_async_copy(
            u_hbm.at[a, pl.ds(r0, p), pl.ds(0, 8)],
            scrA.at[slot, pl.ds(k * p, p)], sems.at[slot, k]))
        cs.append(pltpu.make_async_copy(
            u_hbm.at[a, pl.ds(r0, p), 8],
            scrB.at[slot, 0, pl.ds(k * p, p)], sems.at[slot, _NS + k]))
        cs.append(pltpu.make_async_copy(
            u_hbm.at[a, pl.ds(r0, p), 9],
            scrB.at[slot, 1, pl.ds(k * p, p)], sems.at[slot, 2 * _NS + k]))
    return cs


def _critic_block(u_hbm, act_s_ref, act_o_ref, rw_ref, bias_ref,
                  w1p_ref, w1a_ref, b1_ref, w2_ref, b2_ref, out_ref,
                  scrA, scrB, sems):
    a = pl.program_id(0)
    i = pl.program_id(1)
    na = pl.num_programs(0)
    nb = pl.num_programs(1)
    bB = out_ref.shape[1]
    F = rw_ref.shape[0]
    H = bias_ref.shape[1]
    nact = w2_ref.shape[2]
    step = a * nb + i
    slot = jax.lax.rem(step, 2)

    @pl.when(step == 0)
    def _():
        for c in _copies(u_hbm, scrA, scrB, sems, a, i, slot, bB):
            c.start()

    nstep = step + 1

    @pl.when(nstep < na * nb)
    def _():
        a2 = nstep // nb
        i2 = jax.lax.rem(nstep, nb)
        for c in _copies(u_hbm, scrA, scrB, sems, a2, i2, 1 - slot, bB):
            c.start()

    for c in _copies(u_hbm, scrA, scrB, sems, a, i, slot, bB):
        c.wait()

    x8 = scrA[slot].reshape(bB * 8, F)
    y = jnp.dot(x8, rw_ref[...], preferred_element_type=jnp.float32)
    y3 = y.reshape(bB, 8, 2 * H)
    gmax = jnp.max(y3, axis=1)
    gsum = jnp.sum(y3, axis=1)
    y8 = jnp.dot(scrB[slot, 0], rw_ref[...], preferred_element_type=jnp.float32)
    y9 = jnp.dot(scrB[slot, 1], rw_ref[...], preferred_element_type=jnp.float32)
    mxs = jnp.maximum(jnp.maximum(gmax, y8), y9)
    sms = gsum + y8 + y9
    pooled = jnp.maximum(mxs[:, :H] + sms[:, H:2 * H] + bias_ref[0][None, :],
                         0.0)

    h = (jnp.dot(pooled, w1p_ref[0], preferred_element_type=jnp.float32)
         + jnp.dot(act_o_ref[0], w1a_ref[0], preferred_element_type=jnp.float32)
         + b1_ref[0])
    h = jnp.where(h > 0, h, 0.01 * h)
    q_all = (jnp.dot(h, w2_ref[0], preferred_element_type=jnp.float32)
             + b2_ref[0])

    a_s = act_s_ref[0]
    idx = jnp.argmax(a_s, axis=1)
    onehot = jax.lax.broadcasted_iota(jnp.int32, (bB, nact), 1) == idx[:, None]
    out_ref[0] = jnp.sum(jnp.where(onehot, q_all, 0.0), axis=1, keepdims=True)


def kernel(obs, unary_tensors, actions, edge_index, edge_attr, batch_vec,
           W, root, bias, fc1_w, fc1_b, fc2_w, fc2_b):
    n_agents, B, O, F = unary_tensors.shape
    H = root.shape[1]
    nact = actions.shape[2]
    bB = 512

    wsumd = jnp.sum(W, axis=0) * (1.0 / O)
    rw = jnp.concatenate([root, wsumd], axis=1)
    w1p = fc1_w[:, :H, :]
    w1a = fc1_w[:, H:, :]
    actions_other = actions[::-1]
    bias2 = bias[None, :]
    fc1_b3 = fc1_b[:, None, :]
    fc2_b3 = fc2_b[:, None, :]
    grid = (n_agents, B // bB)

    out = pl.pallas_call(
        _critic_block,
        grid=grid,
        in_specs=[
            pl.BlockSpec(memory_space=pl.ANY),
            pl.BlockSpec((1, bB, nact), lambda a, i: (a, i, 0)),
            pl.BlockSpec((1, bB, nact), lambda a, i: (a, i, 0)),
            pl.BlockSpec((F, 2 * H), lambda a, i: (0, 0)),
            pl.BlockSpec((1, H), lambda a, i: (0, 0)),
            pl.BlockSpec((1, H, H), lambda a, i: (a, 0, 0)),
            pl.BlockSpec((1, nact, H), lambda a, i: (a, 0, 0)),
            pl.BlockSpec((1, 1, H), lambda a, i: (a, 0, 0)),
            pl.BlockSpec((1, H, nact), lambda a, i: (a, 0, 0)),
            pl.BlockSpec((1, 1, nact), lambda a, i: (a, 0, 0)),
        ],
        out_specs=pl.BlockSpec((1, bB, 1), lambda a, i: (a, i, 0)),
        out_shape=jax.ShapeDtypeStruct((n_agents, B, 1), jnp.float32),
        scratch_shapes=[
            pltpu.VMEM((2, bB, 8, F), jnp.float32),
            pltpu.VMEM((2, 2, bB, F), jnp.float32),
            pltpu.SemaphoreType.DMA((2, 3 * _NS)),
        ],
    )(unary_tensors, actions, actions_other, rw, bias2,
      w1p, w1a, fc1_b3, fc2_w, fc2_b3)

    return tuple(out[a] for a in range(n_agents))


# speedup vs baseline: 1.1474x; 1.1474x over previous
import jax
import jax.numpy as jnp
from jax.experimental import pallas as pl
from jax.experimental.pallas import tpu as pltpu

_O = 10


def _copies(u_hbm, scr, sems, a, i, slot, bB):
    return [
        pltpu.make_async_copy(
            u_hbm.at[a, pl.ds(i * bB, bB), o],
            scr.at[slot, o],
            sems.at[slot, o])
        for o in range(_O)
    ]


def _critic_block(u_hbm, act_s_ref, act_o_ref, root_ref, wsumd_ref, bias_ref,
                  w1p_ref, w1a_ref, b1_ref, w2_ref, b2_ref, out_ref,
                  scr, sems):
    a = pl.program_id(0)
    i = pl.program_id(1)
    nb = pl.num_programs(1)
    bB = out_ref.shape[1]
    nact = w2_ref.shape[2]
    slot = jax.lax.rem(i, 2)

    @pl.when(i == 0)
    def _():
        for c in _copies(u_hbm, scr, sems, a, i, slot, bB):
            c.start()

    @pl.when(i + 1 < nb)
    def _():
        for c in _copies(u_hbm, scr, sems, a, i + 1, 1 - slot, bB):
            c.start()

    for c in _copies(u_hbm, scr, sems, a, i, slot, bB):
        c.wait()

    xo = scr[slot, 0]
    acc = xo
    mx = jnp.dot(xo, root_ref[...], preferred_element_type=jnp.float32)
    for o in range(1, _O):
        xo = scr[slot, o]
        acc = acc + xo
        mx = jnp.maximum(
            mx, jnp.dot(xo, root_ref[...], preferred_element_type=jnp.float32))

    mm = jnp.dot(acc, wsumd_ref[...], preferred_element_type=jnp.float32)
    pooled = jnp.maximum(mx + mm + bias_ref[0][None, :], 0.0)

    h = (jnp.dot(pooled, w1p_ref[0], preferred_element_type=jnp.float32)
         + jnp.dot(act_o_ref[0], w1a_ref[0], preferred_element_type=jnp.float32)
         + b1_ref[0])
    h = jnp.where(h > 0, h, 0.01 * h)
    q_all = (jnp.dot(h, w2_ref[0], preferred_element_type=jnp.float32)
             + b2_ref[0])

    a_s = act_s_ref[0]
    idx = jnp.argmax(a_s, axis=1)
    onehot = jax.lax.broadcasted_iota(jnp.int32, (bB, nact), 1) == idx[:, None]
    out_ref[0] = jnp.sum(jnp.where(onehot, q_all, 0.0), axis=1, keepdims=True)


def kernel(obs, unary_tensors, actions, edge_index, edge_attr, batch_vec,
           W, root, bias, fc1_w, fc1_b, fc2_w, fc2_b):
    n_agents, B, O, F = unary_tensors.shape
    H = root.shape[1]
    nact = actions.shape[2]
    bB = 512

    wsumd = jnp.sum(W, axis=0) * (1.0 / O)
    w1p = fc1_w[:, :H, :]
    w1a = fc1_w[:, H:, :]
    actions_other = actions[::-1]
    bias2 = bias[None, :]
    fc1_b3 = fc1_b[:, None, :]
    fc2_b3 = fc2_b[:, None, :]
    grid = (n_agents, B // bB)

    out = pl.pallas_call(
        _critic_block,
        grid=grid,
        in_specs=[
            pl.BlockSpec(memory_space=pl.ANY),
            pl.BlockSpec((1, bB, nact), lambda a, i: (a, i, 0)),
            pl.BlockSpec((1, bB, nact), lambda a, i: (a, i, 0)),
            pl.BlockSpec((F, H), lambda a, i: (0, 0)),
            pl.BlockSpec((F, H), lambda a, i: (0, 0)),
            pl.BlockSpec((1, H), lambda a, i: (0, 0)),
            pl.BlockSpec((1, H, H), lambda a, i: (a, 0, 0)),
            pl.BlockSpec((1, nact, H), lambda a, i: (a, 0, 0)),
            pl.BlockSpec((1, 1, H), lambda a, i: (a, 0, 0)),
            pl.BlockSpec((1, H, nact), lambda a, i: (a, 0, 0)),
            pl.BlockSpec((1, 1, nact), lambda a, i: (a, 0, 0)),
        ],
        out_specs=pl.BlockSpec((1, bB, 1), lambda a, i: (a, i, 0)),
        out_shape=jax.ShapeDtypeStruct((n_agents, B, 1), jnp.float32),
        scratch_shapes=[
            pltpu.VMEM((2, _O, bB, F), jnp.float32),
            pltpu.SemaphoreType.DMA((2, _O)),
        ],
        compiler_params=pltpu.CompilerParams(
            dimension_semantics=("parallel", "arbitrary")),
    )(unary_tensors, actions, actions_other, root, wsumd, bias2,
      w1p, w1a, fc1_b3, fc2_w, fc2_b3)

    return tuple(out[a] for a in range(n_agents))


# speedup vs baseline: 1.2126x; 1.0568x over previous
import jax
import jax.numpy as jnp
from jax.experimental import pallas as pl
from jax.experimental.pallas import tpu as pltpu

_O = 10


def _copies(u_hbm, scr, sems, a, i, slot, bB):
    return [
        pltpu.make_async_copy(
            u_hbm.at[a, pl.ds(i * bB, bB), o],
            scr.at[slot, o],
            sems.at[slot, o])
        for o in range(_O)
    ]


def _critic_block(u_hbm, act_s_ref, act_o_ref, root_ref, wsumd_ref, bias_ref,
                  w1p_ref, w1a_ref, b1_ref, w2_ref, b2_ref, out_ref,
                  scr, sems):
    a = pl.program_id(0)
    i = pl.program_id(1)
    nb = pl.num_programs(1)
    bB = out_ref.shape[1]
    nact = w2_ref.shape[2]
    slot = jax.lax.rem(i, 2)

    @pl.when(i == 0)
    def _():
        for c in _copies(u_hbm, scr, sems, a, i, slot, bB):
            c.start()

    @pl.when(i + 1 < nb)
    def _():
        for c in _copies(u_hbm, scr, sems, a, i + 1, 1 - slot, bB):
            c.start()

    for c in _copies(u_hbm, scr, sems, a, i, slot, bB):
        c.wait()

    xo = scr[slot, 0]
    acc = xo
    mx = jnp.dot(xo, root_ref[...], preferred_element_type=jnp.float32)
    for o in range(1, _O):
        xo = scr[slot, o]
        acc = acc + xo
        mx = jnp.maximum(
            mx, jnp.dot(xo, root_ref[...], preferred_element_type=jnp.float32))

    mm = jnp.dot(acc, wsumd_ref[...], preferred_element_type=jnp.float32)
    pooled = jnp.maximum(mx + mm + bias_ref[0][None, :], 0.0)

    h = (jnp.dot(pooled, w1p_ref[0], preferred_element_type=jnp.float32)
         + jnp.dot(act_o_ref[0], w1a_ref[0], preferred_element_type=jnp.float32)
         + b1_ref[0])
    h = jnp.where(h > 0, h, 0.01 * h)
    q_all = (jnp.dot(h, w2_ref[0], preferred_element_type=jnp.float32)
             + b2_ref[0])

    a_s = act_s_ref[0]
    idx = jnp.argmax(a_s, axis=1)
    onehot = jax.lax.broadcasted_iota(jnp.int32, (bB, nact), 1) == idx[:, None]
    out_ref[0] = jnp.sum(jnp.where(onehot, q_all, 0.0), axis=1, keepdims=True)


def kernel(obs, unary_tensors, actions, edge_index, edge_attr, batch_vec,
           W, root, bias, fc1_w, fc1_b, fc2_w, fc2_b):
    n_agents, B, O, F = unary_tensors.shape
    H = root.shape[1]
    nact = actions.shape[2]
    bB = 1024

    wsumd = jnp.sum(W, axis=0) * (1.0 / O)
    w1p = fc1_w[:, :H, :]
    w1a = fc1_w[:, H:, :]
    actions_other = actions[::-1]
    bias2 = bias[None, :]
    fc1_b3 = fc1_b[:, None, :]
    fc2_b3 = fc2_b[:, None, :]
    grid = (n_agents, B // bB)

    out = pl.pallas_call(
        _critic_block,
        grid=grid,
        in_specs=[
            pl.BlockSpec(memory_space=pl.ANY),
            pl.BlockSpec((1, bB, nact), lambda a, i: (a, i, 0)),
            pl.BlockSpec((1, bB, nact), lambda a, i: (a, i, 0)),
            pl.BlockSpec((F, H), lambda a, i: (0, 0)),
            pl.BlockSpec((F, H), lambda a, i: (0, 0)),
            pl.BlockSpec((1, H), lambda a, i: (0, 0)),
            pl.BlockSpec((1, H, H), lambda a, i: (a, 0, 0)),
            pl.BlockSpec((1, nact, H), lambda a, i: (a, 0, 0)),
            pl.BlockSpec((1, 1, H), lambda a, i: (a, 0, 0)),
            pl.BlockSpec((1, H, nact), lambda a, i: (a, 0, 0)),
            pl.BlockSpec((1, 1, nact), lambda a, i: (a, 0, 0)),
        ],
        out_specs=pl.BlockSpec((1, bB, 1), lambda a, i: (a, i, 0)),
        out_shape=jax.ShapeDtypeStruct((n_agents, B, 1), jnp.float32),
        scratch_shapes=[
            pltpu.VMEM((2, _O, bB, F), jnp.float32),
            pltpu.SemaphoreType.DMA((2, _O)),
        ],
        compiler_params=pltpu.CompilerParams(
            dimension_semantics=("parallel", "arbitrary")),
    )(unary_tensors, actions, actions_other, root, wsumd, bias2,
      w1p, w1a, fc1_b3, fc2_w, fc2_b3)

    return tuple(out[a] for a in range(n_agents))


# speedup vs baseline: 1.2208x; 1.0067x over previous
import jax
import jax.numpy as jnp
from jax.experimental import pallas as pl
from jax.experimental.pallas import tpu as pltpu

_O = 10


def _copies(u_hbm, scr, sems, a, i, slot, bB):
    return [
        pltpu.make_async_copy(
            u_hbm.at[a, pl.ds(i * bB, bB), o],
            scr.at[slot, o],
            sems.at[slot, o])
        for o in range(_O)
    ]


def _critic_block(u_hbm, act_s_ref, act_o_ref, root_ref, wsumd_ref, bias_ref,
                  w1p_ref, w1a_ref, b1_ref, w2_ref, b2_ref, out_ref,
                  scr, sems):
    a = pl.program_id(0)
    i = pl.program_id(1)
    nb = pl.num_programs(1)
    bB = out_ref.shape[1]
    nact = w2_ref.shape[2]
    slot = jax.lax.rem(i, 2)

    @pl.when(i == 0)
    def _():
        for c in _copies(u_hbm, scr, sems, a, i, slot, bB):
            c.start()

    @pl.when(i + 1 < nb)
    def _():
        for c in _copies(u_hbm, scr, sems, a, i + 1, 1 - slot, bB):
            c.start()

    for c in _copies(u_hbm, scr, sems, a, i, slot, bB):
        c.wait()

    xo = scr[slot, 0]
    acc = xo
    mx = jnp.dot(xo, root_ref[...], preferred_element_type=jnp.float32)
    for o in range(1, _O):
        xo = scr[slot, o]
        acc = acc + xo
        mx = jnp.maximum(
            mx, jnp.dot(xo, root_ref[...], preferred_element_type=jnp.float32))

    mm = jnp.dot(acc, wsumd_ref[...], preferred_element_type=jnp.float32)
    pooled = jnp.maximum(mx + mm + bias_ref[0][None, :], 0.0)

    h = (jnp.dot(pooled, w1p_ref[0], preferred_element_type=jnp.float32)
         + jnp.dot(act_o_ref[0], w1a_ref[0], preferred_element_type=jnp.float32)
         + b1_ref[0])
    h = jnp.where(h > 0, h, 0.01 * h)
    q_all = (jnp.dot(h, w2_ref[0], preferred_element_type=jnp.float32)
             + b2_ref[0])

    a_s = act_s_ref[0]
    idx = jnp.argmax(a_s, axis=1)
    onehot = jax.lax.broadcasted_iota(jnp.int32, (bB, nact), 1) == idx[:, None]
    out_ref[0] = jnp.sum(jnp.where(onehot, q_all, 0.0), axis=1, keepdims=True)


def kernel(obs, unary_tensors, actions, edge_index, edge_attr, batch_vec,
           W, root, bias, fc1_w, fc1_b, fc2_w, fc2_b):
    n_agents, B, O, F = unary_tensors.shape
    H = root.shape[1]
    nact = actions.shape[2]
    bB = 2048

    wsumd = jnp.sum(W, axis=0) * (1.0 / O)
    w1p = fc1_w[:, :H, :]
    w1a = fc1_w[:, H:, :]
    actions_other = actions[::-1]
    bias2 = bias[None, :]
    fc1_b3 = fc1_b[:, None, :]
    fc2_b3 = fc2_b[:, None, :]
    grid = (n_agents, B // bB)

    out = pl.pallas_call(
        _critic_block,
        grid=grid,
        in_specs=[
            pl.BlockSpec(memory_space=pl.ANY),
            pl.BlockSpec((1, bB, nact), lambda a, i: (a, i, 0)),
            pl.BlockSpec((1, bB, nact), lambda a, i: (a, i, 0)),
            pl.BlockSpec((F, H), lambda a, i: (0, 0)),
            pl.BlockSpec((F, H), lambda a, i: (0, 0)),
            pl.BlockSpec((1, H), lambda a, i: (0, 0)),
            pl.BlockSpec((1, H, H), lambda a, i: (a, 0, 0)),
            pl.BlockSpec((1, nact, H), lambda a, i: (a, 0, 0)),
            pl.BlockSpec((1, 1, H), lambda a, i: (a, 0, 0)),
            pl.BlockSpec((1, H, nact), lambda a, i: (a, 0, 0)),
            pl.BlockSpec((1, 1, nact), lambda a, i: (a, 0, 0)),
        ],
        out_specs=pl.BlockSpec((1, bB, 1), lambda a, i: (a, i, 0)),
        out_shape=jax.ShapeDtypeStruct((n_agents, B, 1), jnp.float32),
        scratch_shapes=[
            pltpu.VMEM((2, _O, bB, F), jnp.float32),
            pltpu.SemaphoreType.DMA((2, _O)),
        ],
        compiler_params=pltpu.CompilerParams(
            dimension_semantics=("parallel", "arbitrary")),
    )(unary_tensors, actions, actions_other, root, wsumd, bias2,
      w1p, w1a, fc1_b3, fc2_w, fc2_b3)

    return tuple(out[a] for a in range(n_agents))
